# baseline (speedup 1.0000x reference)
import functools

import jax
import jax.numpy as jnp
from jax import lax
from jax.experimental import pallas as pl
from jax.experimental.pallas import tpu as pltpu
from jax.experimental.pallas import tpu_sc as plsc

E = 64
K = 2
H = 1024
I = 1024
T = 2048
TILE = 128
NT_MAX = 96
PS = NT_MAX * TILE
NCHUNK = 32
CTOK = T // NCHUNK

_i32 = jnp.int32
_f32 = jnp.float32


def _router_body(x_ref, gw_ref, e0_ref, e1_ref, w0_ref, w1_ref):
    x = x_ref[...]
    gw = gw_ref[...]
    logits = lax.dot_general(x, gw, (((1,), (1,)), ((), ())),
                             preferred_element_type=_f32)
    idx = lax.broadcasted_iota(_i32, logits.shape, 1)
    m0 = jnp.max(logits, axis=1, keepdims=True)
    a0 = jnp.min(jnp.where(logits == m0, idx, E), axis=1, keepdims=True)
    masked = jnp.where(idx == a0, -jnp.inf, logits)
    m1 = jnp.max(masked, axis=1, keepdims=True)
    a1 = jnp.min(jnp.where(masked == m1, idx, E), axis=1, keepdims=True)
    r = jnp.exp(m1 - m0)
    w0 = 1.0 / (1.0 + r)
    w1 = r / (1.0 + r)
    e0_ref[...] = a0
    e1_ref[...] = a1
    w0_ref[...] = w0
    w1_ref[...] = w1


def _router(x, gate_weight):
    return pl.pallas_call(
        _router_body,
        out_shape=[
            jax.ShapeDtypeStruct((T, 1), _i32),
            jax.ShapeDtypeStruct((T, 1), _i32),
            jax.ShapeDtypeStruct((T, 1), _f32),
            jax.ShapeDtypeStruct((T, 1), _f32),
        ],
    )(x, gate_weight)


def _lane():
    return lax.iota(_i32, 16)


def _sread(ref, i):
    return ref[pl.ds(i, 16)][0]


def _dispatch_body(e0_hbm, e1_hbm, x_hbm,
                   hg_hbm, posa_hbm, posb_hbm, te_hbm, ntv_hbm, lcg_hbm,
                   e0c_ref, e1c_ref, hist_ref, lc_ref,
                   posa_ref, posb_ref, xrows_ref, te_ref, ntv_ref,
                   hist_sm, cnt_sm, run_sm, te_sm,
                   sem):
    ci = lax.axis_index("c")
    si = lax.axis_index("s")
    lane = _lane()

    for dc in range(2):
        cc = 2 * si + dc

        def zbody(e, _):
            hist_sm[e] = 0
            return 0
        lax.fori_loop(0, E, zbody, 0)
        pltpu.sync_copy(e0_hbm.at[pl.ds(cc * CTOK, CTOK)],
                        e0c_ref.at[pl.ds(0, CTOK)])
        pltpu.sync_copy(e1_hbm.at[pl.ds(cc * CTOK, CTOK)],
                        e1c_ref.at[pl.ds(0, CTOK)])

        def abody(j, _):
            ea = _sread(e0c_ref, j)
            hist_sm[ea] = hist_sm[ea] + 1
            eb = _sread(e1c_ref, j)
            hist_sm[eb] = hist_sm[eb] + 1
            return 0
        lax.fori_loop(0, CTOK, abody, 0)
        for v in range(4):
            acc = jnp.zeros((16,), _i32)
            for l in range(16):
                acc = jnp.where(lane == l, hist_sm[v * 16 + l], acc)
            hist_ref[pl.ds(v * 16, 16)] = acc
        pltpu.sync_copy(hist_ref, lcg_hbm.at[ci, cc])

    plsc.subcore_barrier()
    pltpu.sync_copy(lcg_hbm.at[ci], lc_ref)

    c = 2 * si + ci
    for v in range(4):
        sl = pl.ds(v * 16, 16)

        def pb(cx, acc):
            return acc + lc_ref[cx, sl]
        pre = lax.fori_loop(0, c, pb, jnp.zeros((16,), _i32))
        tot = lax.fori_loop(0, NCHUNK, pb, jnp.zeros((16,), _i32))
        for l in range(16):
            run_sm[v * 16 + l] = pre[l]
            cnt_sm[v * 16 + l] = tot[l]

    def pstart_body(e, acc):
        run_sm[e] = run_sm[e] + acc
        ce = cnt_sm[e]
        return acc + ((ce + TILE - 1) // TILE) * TILE
    lax.fori_loop(0, E, pstart_body, 0)

    pltpu.sync_copy(e0_hbm.at[pl.ds(c * CTOK, CTOK)],
                    e0c_ref.at[pl.ds(0, CTOK)])
    pltpu.sync_copy(e1_hbm.at[pl.ds(c * CTOK, CTOK)],
                    e1c_ref.at[pl.ds(0, CTOK)])
    pltpu.sync_copy(x_hbm.at[pl.ds(c * CTOK, CTOK)], xrows_ref)

    for q in range(CTOK // 16):
        veca = jnp.zeros((16,), _i32)
        vecb = jnp.zeros((16,), _i32)
        ea16 = e0c_ref[pl.ds(q * 16, 16)]
        eb16 = e1c_ref[pl.ds(q * 16, 16)]
        for l in range(16):
            ea = ea16[l]
            pa = run_sm[ea]
            run_sm[ea] = pa + 1
            veca = jnp.where(lane == l, pa, veca)
            eb = eb16[l]
            pb_ = run_sm[eb]
            run_sm[eb] = pb_ + 1
            vecb = jnp.where(lane == l, pb_, vecb)
        posa_ref[pl.ds(q * 16, 16)] = veca
        posb_ref[pl.ds(q * 16, 16)] = vecb

    pltpu.sync_copy(posa_ref, posa_hbm.at[pl.ds(c * CTOK, CTOK)])
    pltpu.sync_copy(posb_ref, posb_hbm.at[pl.ds(c * CTOK, CTOK)])
    da = pltpu.async_copy(xrows_ref, hg_hbm.at[posa_ref], sem)
    db = pltpu.async_copy(xrows_ref, hg_hbm.at[posb_ref], sem)
    da.wait()
    db.wait()

    @pl.when((ci == 0) & (si == 0))
    def _():
        def dbody(e, k):
            nte = (cnt_sm[e] + TILE - 1) // TILE

            def inner(r, _):
                te_sm[k + r] = e
                return 0
            lax.fori_loop(0, nte, inner, 0)
            return k + nte
        nt = lax.fori_loop(0, E, dbody, 0)
        last = te_sm[nt - 1]

        def tbody(k, _):
            te_sm[k] = last
            return 0
        lax.fori_loop(nt, 128, tbody, 0)
        for v in range(8):
            acc = jnp.zeros((16,), _i32)
            for l in range(16):
                acc = jnp.where(lane == l, te_sm[v * 16 + l], acc)
            te_ref[pl.ds(v * 16, 16)] = acc
        ntv_ref[...] = jnp.where(lane == 0, nt, 0)
        pltpu.sync_copy(te_ref, te_hbm)
        pltpu.sync_copy(ntv_ref, ntv_hbm)


def _dispatch(e0, e1, x):
    mesh = plsc.VectorSubcoreMesh(core_axis_name="c", subcore_axis_name="s")
    f = pl.kernel(
        _dispatch_body,
        out_type=[
            jax.ShapeDtypeStruct((PS, H), _f32),
            jax.ShapeDtypeStruct((T,), _i32),
            jax.ShapeDtypeStruct((T,), _i32),
            jax.ShapeDtypeStruct((128,), _i32),
            jax.ShapeDtypeStruct((16,), _i32),
            jax.ShapeDtypeStruct((2, NCHUNK, E), _i32),
        ],
        mesh=mesh,
        scratch_types=[
            pltpu.VMEM((CTOK + 16,), _i32),
            pltpu.VMEM((CTOK + 16,), _i32),
            pltpu.VMEM((E,), _i32),
            pltpu.VMEM((NCHUNK, E), _i32),
            pltpu.VMEM((CTOK,), _i32),
            pltpu.VMEM((CTOK,), _i32),
            pltpu.VMEM((CTOK, H), _f32),
            pltpu.VMEM((128,), _i32),
            pltpu.VMEM((16,), _i32),
            pltpu.SMEM((E,), _i32),
            pltpu.SMEM((E,), _i32),
            pltpu.SMEM((E,), _i32),
            pltpu.SMEM((128,), _i32),
            pltpu.SemaphoreType.DMA,
        ],
    )
    return f(e0, e1, x)


def _mlp_body(te_ref, nt_ref, hg_ref, fc_ref, pj_ref, out_ref):
    i = pl.program_id(0)

    @pl.when(i < nt_ref[0])
    def _():
        h = hg_ref[...]
        a = lax.dot_general(h, fc_ref[0], (((1,), (1,)), ((), ())),
                            preferred_element_type=_f32)
        a = 0.5 * a * (1.0 + lax.erf(a * 0.7071067811865476))
        out_ref[...] = lax.dot_general(a, pj_ref[0], (((1,), (1,)), ((), ())),
                                       preferred_element_type=_f32)


def _mlp(te, ntv, hg, c_fc_weight, c_proj_weight):
    grid_spec = pltpu.PrefetchScalarGridSpec(
        num_scalar_prefetch=2,
        grid=(NT_MAX,),
        in_specs=[
            pl.BlockSpec((TILE, H),
                         lambda i, te, nt: (jnp.minimum(i, nt[0] - 1), 0)),
            pl.BlockSpec((1, I, H), lambda i, te, nt: (te[i], 0, 0)),
            pl.BlockSpec((1, H, I), lambda i, te, nt: (te[i], 0, 0)),
        ],
        out_specs=pl.BlockSpec(
            (TILE, H), lambda i, te, nt: (jnp.minimum(i, nt[0] - 1), 0)),
    )
    return pl.pallas_call(
        _mlp_body,
        grid_spec=grid_spec,
        out_shape=jax.ShapeDtypeStruct((PS, H), _f32),
        compiler_params=pltpu.CompilerParams(
            dimension_semantics=("arbitrary",)),
    )(te, ntv, hg, c_fc_weight, c_proj_weight)


def _combine_body(yg_hbm, posa_hbm, posb_hbm, w0_hbm, w1_hbm, out_hbm,
                  pa_ref, pb_ref, w0_ref, w1_ref, ra_ref, rb_ref, sem):
    ci = lax.axis_index("c")
    si = lax.axis_index("s")
    w = 2 * si + ci
    SUB = 32
    for half in range(2):
        base = w * 64 + half * SUB
        pltpu.sync_copy(posa_hbm.at[pl.ds(base, SUB)], pa_ref)
        pltpu.sync_copy(posb_hbm.at[pl.ds(base, SUB)], pb_ref)
        pltpu.sync_copy(w0_hbm.at[pl.ds(base, SUB)], w0_ref.at[pl.ds(0, SUB)])
        pltpu.sync_copy(w1_hbm.at[pl.ds(base, SUB)], w1_ref.at[pl.ds(0, SUB)])
        ca = pltpu.async_copy(yg_hbm.at[pa_ref], ra_ref, sem)
        cb = pltpu.async_copy(yg_hbm.at[pb_ref], rb_ref, sem)
        ca.wait()
        cb.wait()

        def tbody(t, _):
            ga = _sread(w0_ref, t)
            gb = _sread(w1_ref, t)

            def vbody(v, _):
                sl = pl.ds(v * 16, 16)
                ra_ref[t, sl] = ga * ra_ref[t, sl] + gb * rb_ref[t, sl]
                return 0
            lax.fori_loop(0, H // 16, vbody, 0)
            return 0
        lax.fori_loop(0, SUB, tbody, 0)
        pltpu.sync_copy(ra_ref, out_hbm.at[pl.ds(base, SUB)])


def _combine(yg, posa, posb, w0, w1):
    mesh = plsc.VectorSubcoreMesh(core_axis_name="c", subcore_axis_name="s")
    f = pl.kernel(
        _combine_body,
        out_type=jax.ShapeDtypeStruct((T, H), _f32),
        mesh=mesh,
        scratch_types=[
            pltpu.VMEM((32,), _i32),
            pltpu.VMEM((32,), _i32),
            pltpu.VMEM((32 + 16,), _f32),
            pltpu.VMEM((32 + 16,), _f32),
            pltpu.VMEM((32, H), _f32),
            pltpu.VMEM((32, H), _f32),
            pltpu.SemaphoreType.DMA,
        ],
    )
    return f(yg, posa, posb, w0, w1)


def kernel(hidden_states, gate_weight, c_fc_weight, c_proj_weight):
    B, S, Hh = hidden_states.shape
    x = hidden_states.reshape(T, H)
    e0b, e1b, w0b, w1b = _router(x, gate_weight)
    e0 = e0b.reshape(T)
    e1 = e1b.reshape(T)
    w0 = w0b.reshape(T)
    w1 = w1b.reshape(T)
    hg, posa, posb, te, ntv, _ = _dispatch(e0, e1, x)
    yg = _mlp(te, ntv, hg, c_fc_weight, c_proj_weight)
    out = _combine(yg, posa, posb, w0, w1)
    return out.reshape(B, S, Hh)


# speedup vs baseline: 1.0119x; 1.0119x over previous
import functools

import jax
import jax.numpy as jnp
from jax import lax
from jax.experimental import pallas as pl
from jax.experimental.pallas import tpu as pltpu
from jax.experimental.pallas import tpu_sc as plsc

E = 64
K = 2
H = 1024
I = 1024
T = 2048
TILE = 128
NT_MAX = 96
PS = NT_MAX * TILE
NCHUNK = 32
CTOK = T // NCHUNK

_i32 = jnp.int32
_f32 = jnp.float32


def _router_body(x_ref, gw_ref, e0_ref, e1_ref, w0_ref, w1_ref):
    x = x_ref[...]
    gw = gw_ref[...]
    logits = lax.dot_general(x, gw, (((1,), (1,)), ((), ())),
                             preferred_element_type=_f32)
    idx = lax.broadcasted_iota(_i32, logits.shape, 1)
    m0 = jnp.max(logits, axis=1, keepdims=True)
    a0 = jnp.min(jnp.where(logits == m0, idx, E), axis=1, keepdims=True)
    masked = jnp.where(idx == a0, -jnp.inf, logits)
    m1 = jnp.max(masked, axis=1, keepdims=True)
    a1 = jnp.min(jnp.where(masked == m1, idx, E), axis=1, keepdims=True)
    r = jnp.exp(m1 - m0)
    w0 = 1.0 / (1.0 + r)
    w1 = r / (1.0 + r)
    e0_ref[...] = a0
    e1_ref[...] = a1
    w0_ref[...] = w0
    w1_ref[...] = w1


def _router(x, gate_weight):
    return pl.pallas_call(
        _router_body,
        out_shape=[
            jax.ShapeDtypeStruct((T, 1), _i32),
            jax.ShapeDtypeStruct((T, 1), _i32),
            jax.ShapeDtypeStruct((T, 1), _f32),
            jax.ShapeDtypeStruct((T, 1), _f32),
        ],
    )(x, gate_weight)


def _lane():
    return lax.iota(_i32, 16)


def _sread(ref, i):
    return ref[pl.ds(i, 16)][0]


def _dispatch_body(e0_hbm, e1_hbm, x_hbm,
                   hg_hbm, posa_hbm, posb_hbm, te_hbm, ntv_hbm, lcg_hbm,
                   e0c_ref, e1c_ref, hist_ref, lc_ref,
                   posa_ref, posb_ref, xrows_ref, te_ref, ntv_ref,
                   hist_sm, cnt_sm, run_sm, te_sm,
                   sem):
    ci = lax.axis_index("c")
    si = lax.axis_index("s")
    lane = _lane()
    c = 2 * si + ci

    dx = pltpu.async_copy(x_hbm.at[pl.ds(c * CTOK, CTOK)], xrows_ref, sem)

    for dc in range(2):
        cc = 2 * si + (1 - ci) if dc == 0 else c

        def zbody(e, _):
            hist_sm[e] = 0
            return 0
        lax.fori_loop(0, E, zbody, 0)
        pltpu.sync_copy(e0_hbm.at[pl.ds(cc * CTOK, CTOK)],
                        e0c_ref.at[pl.ds(0, CTOK)])
        pltpu.sync_copy(e1_hbm.at[pl.ds(cc * CTOK, CTOK)],
                        e1c_ref.at[pl.ds(0, CTOK)])

        def abody(j, _):
            ea = _sread(e0c_ref, j)
            hist_sm[ea] = hist_sm[ea] + 1
            eb = _sread(e1c_ref, j)
            hist_sm[eb] = hist_sm[eb] + 1
            return 0
        lax.fori_loop(0, CTOK, abody, 0)
        for v in range(4):
            acc = jnp.zeros((16,), _i32)
            for l in range(16):
                acc = jnp.where(lane == l, hist_sm[v * 16 + l], acc)
            hist_ref[pl.ds(v * 16, 16)] = acc
        pltpu.sync_copy(hist_ref, lcg_hbm.at[ci, cc])

    plsc.subcore_barrier()
    pltpu.sync_copy(lcg_hbm.at[ci], lc_ref)

    for v in range(4):
        sl = pl.ds(v * 16, 16)

        def pb(cx, acc):
            return acc + lc_ref[cx, sl]
        pre = lax.fori_loop(0, c, pb, jnp.zeros((16,), _i32))
        tot = lax.fori_loop(0, NCHUNK, pb, jnp.zeros((16,), _i32))
        for l in range(16):
            run_sm[v * 16 + l] = pre[l]
            cnt_sm[v * 16 + l] = tot[l]

    def pstart_body(e, acc):
        run_sm[e] = run_sm[e] + acc
        ce = cnt_sm[e]
        return acc + ((ce + TILE - 1) // TILE) * TILE
    lax.fori_loop(0, E, pstart_body, 0)

    for q in range(CTOK // 16):
        veca = jnp.zeros((16,), _i32)
        vecb = jnp.zeros((16,), _i32)
        ea16 = e0c_ref[pl.ds(q * 16, 16)]
        eb16 = e1c_ref[pl.ds(q * 16, 16)]
        for l in range(16):
            ea = ea16[l]
            pa = run_sm[ea]
            run_sm[ea] = pa + 1
            veca = jnp.where(lane == l, pa, veca)
            eb = eb16[l]
            pb_ = run_sm[eb]
            run_sm[eb] = pb_ + 1
            vecb = jnp.where(lane == l, pb_, vecb)
        posa_ref[pl.ds(q * 16, 16)] = veca
        posb_ref[pl.ds(q * 16, 16)] = vecb

    pltpu.sync_copy(posa_ref, posa_hbm.at[pl.ds(c * CTOK, CTOK)])
    pltpu.sync_copy(posb_ref, posb_hbm.at[pl.ds(c * CTOK, CTOK)])
    dx.wait()
    da = pltpu.async_copy(xrows_ref, hg_hbm.at[posa_ref], sem)
    db = pltpu.async_copy(xrows_ref, hg_hbm.at[posb_ref], sem)
    da.wait()
    db.wait()

    @pl.when((ci == 0) & (si == 0))
    def _():
        def dbody(e, k):
            nte = (cnt_sm[e] + TILE - 1) // TILE

            def inner(r, _):
                te_sm[k + r] = e
                return 0
            lax.fori_loop(0, nte, inner, 0)
            return k + nte
        nt = lax.fori_loop(0, E, dbody, 0)
        last = te_sm[nt - 1]

        def tbody(k, _):
            te_sm[k] = last
            return 0
        lax.fori_loop(nt, 128, tbody, 0)
        for v in range(8):
            acc = jnp.zeros((16,), _i32)
            for l in range(16):
                acc = jnp.where(lane == l, te_sm[v * 16 + l], acc)
            te_ref[pl.ds(v * 16, 16)] = acc
        ntv_ref[...] = jnp.where(lane == 0, nt, 0)
        pltpu.sync_copy(te_ref, te_hbm)
        pltpu.sync_copy(ntv_ref, ntv_hbm)


def _dispatch(e0, e1, x):
    mesh = plsc.VectorSubcoreMesh(core_axis_name="c", subcore_axis_name="s")
    f = pl.kernel(
        _dispatch_body,
        out_type=[
            jax.ShapeDtypeStruct((PS, H), _f32),
            jax.ShapeDtypeStruct((T,), _i32),
            jax.ShapeDtypeStruct((T,), _i32),
            jax.ShapeDtypeStruct((128,), _i32),
            jax.ShapeDtypeStruct((16,), _i32),
            jax.ShapeDtypeStruct((2, NCHUNK, E), _i32),
        ],
        mesh=mesh,
        scratch_types=[
            pltpu.VMEM((CTOK + 16,), _i32),
            pltpu.VMEM((CTOK + 16,), _i32),
            pltpu.VMEM((E,), _i32),
            pltpu.VMEM((NCHUNK, E), _i32),
            pltpu.VMEM((CTOK,), _i32),
            pltpu.VMEM((CTOK,), _i32),
            pltpu.VMEM((CTOK, H), _f32),
            pltpu.VMEM((128,), _i32),
            pltpu.VMEM((16,), _i32),
            pltpu.SMEM((E,), _i32),
            pltpu.SMEM((E,), _i32),
            pltpu.SMEM((E,), _i32),
            pltpu.SMEM((128,), _i32),
            pltpu.SemaphoreType.DMA,
        ],
    )
    return f(e0, e1, x)


def _mlp_body(te_ref, nt_ref, hg_ref, fc_ref, pj_ref, out_ref):
    i = pl.program_id(0)

    @pl.when(i < nt_ref[0])
    def _():
        h = hg_ref[...]
        a = lax.dot_general(h, fc_ref[0], (((1,), (1,)), ((), ())),
                            preferred_element_type=_f32)
        a = 0.5 * a * (1.0 + lax.erf(a * 0.7071067811865476))
        out_ref[...] = lax.dot_general(a, pj_ref[0], (((1,), (1,)), ((), ())),
                                       preferred_element_type=_f32)


def _mlp(te, ntv, hg, c_fc_weight, c_proj_weight):
    grid_spec = pltpu.PrefetchScalarGridSpec(
        num_scalar_prefetch=2,
        grid=(NT_MAX,),
        in_specs=[
            pl.BlockSpec((TILE, H),
                         lambda i, te, nt: (jnp.minimum(i, nt[0] - 1), 0)),
            pl.BlockSpec((1, I, H), lambda i, te, nt: (te[i], 0, 0)),
            pl.BlockSpec((1, H, I), lambda i, te, nt: (te[i], 0, 0)),
        ],
        out_specs=pl.BlockSpec(
            (TILE, H), lambda i, te, nt: (jnp.minimum(i, nt[0] - 1), 0)),
    )
    return pl.pallas_call(
        _mlp_body,
        grid_spec=grid_spec,
        out_shape=jax.ShapeDtypeStruct((PS, H), _f32),
        compiler_params=pltpu.CompilerParams(
            dimension_semantics=("arbitrary",)),
    )(te, ntv, hg, c_fc_weight, c_proj_weight)


def _combine_body(yg_hbm, posa_hbm, posb_hbm, w0_hbm, w1_hbm, out_hbm,
                  pa_ref, pb_ref, w0_ref, w1_ref, ra_ref, rb_ref, sem):
    ci = lax.axis_index("c")
    si = lax.axis_index("s")
    w = 2 * si + ci
    SUB = 32
    for half in range(2):
        base = w * 64 + half * SUB
        pltpu.sync_copy(posa_hbm.at[pl.ds(base, SUB)], pa_ref)
        pltpu.sync_copy(posb_hbm.at[pl.ds(base, SUB)], pb_ref)
        pltpu.sync_copy(w0_hbm.at[pl.ds(base, SUB)], w0_ref.at[pl.ds(0, SUB)])
        pltpu.sync_copy(w1_hbm.at[pl.ds(base, SUB)], w1_ref.at[pl.ds(0, SUB)])
        ca = pltpu.async_copy(yg_hbm.at[pa_ref], ra_ref, sem)
        cb = pltpu.async_copy(yg_hbm.at[pb_ref], rb_ref, sem)
        ca.wait()
        cb.wait()

        def tbody(t, _):
            ga = _sread(w0_ref, t)
            gb = _sread(w1_ref, t)

            def vbody(v, _):
                sl = pl.ds(v * 16, 16)
                ra_ref[t, sl] = ga * ra_ref[t, sl] + gb * rb_ref[t, sl]
                return 0
            lax.fori_loop(0, H // 16, vbody, 0)
            return 0
        lax.fori_loop(0, SUB, tbody, 0)
        pltpu.sync_copy(ra_ref, out_hbm.at[pl.ds(base, SUB)])


def _combine(yg, posa, posb, w0, w1):
    mesh = plsc.VectorSubcoreMesh(core_axis_name="c", subcore_axis_name="s")
    f = pl.kernel(
        _combine_body,
        out_type=jax.ShapeDtypeStruct((T, H), _f32),
        mesh=mesh,
        scratch_types=[
            pltpu.VMEM((32,), _i32),
            pltpu.VMEM((32,), _i32),
            pltpu.VMEM((32 + 16,), _f32),
            pltpu.VMEM((32 + 16,), _f32),
            pltpu.VMEM((32, H), _f32),
            pltpu.VMEM((32, H), _f32),
            pltpu.SemaphoreType.DMA,
        ],
    )
    return f(yg, posa, posb, w0, w1)


def kernel(hidden_states, gate_weight, c_fc_weight, c_proj_weight):
    B, S, Hh = hidden_states.shape
    x = hidden_states.reshape(T, H)
    e0b, e1b, w0b, w1b = _router(x, gate_weight)
    e0 = e0b.reshape(T)
    e1 = e1b.reshape(T)
    w0 = w0b.reshape(T)
    w1 = w1b.reshape(T)
    hg, posa, posb, te, ntv, _ = _dispatch(e0, e1, x)
    yg = _mlp(te, ntv, hg, c_fc_weight, c_proj_weight)
    out = _combine(yg, posa, posb, w0, w1)
    return out.reshape(B, S, Hh)
